# Initial kernel scaffold; baseline (speedup 1.0000x reference)
import jax
import jax.numpy as jnp
from jax.experimental import pallas as pl


def kernel(sinogram, bp_grid, mask_idx, scale, center):
    raise NotImplementedError("write your pallas kernel here")


# speedup vs baseline: 4.4146x; 4.4146x over previous
import functools

import jax
import jax.numpy as jnp
from jax import lax
from jax.experimental import pallas as pl
from jax.experimental.pallas import tpu as pltpu
from jax.experimental.pallas import tpu_sc as plsc

DET = 768
VIEW = 512
SP = 256
PAD = 64
W_IN = DET + 2 * PAD
W_PAD = W_IN + 8
V_PAD = VIEW + 2
NC, NS = 2, 16
NW = NC * NS
LANES = 16


def _sc_body(p_hbm, c_hbm, out_hbm, cbuf, win, obuf):
    wid = lax.axis_index("s") * NC + lax.axis_index("c")
    b = wid // 8
    v0 = (wid % 8) * 64
    lane = lax.iota(jnp.int32, LANES)

    pltpu.sync_copy(
        c_hbm.at[pl.ds(pl.multiple_of(b * VIEW + v0, 64), 64)], cbuf)

    for i in range(2):
        row = b * V_PAD + v0 + i
        slot = lax.rem(v0 + i, 3)
        pltpu.sync_copy(
            p_hbm.at[pl.ds(pl.multiple_of(row * W_PAD, 8), W_PAD)],
            win.at[pl.ds(pl.multiple_of(slot * W_PAD, 8), W_PAD)])

    def body(t, carry):
        v = v0 + t
        cv = plsc.load_gather(cbuf, [jnp.full((LANES,), t, jnp.int32)])
        w0 = cv - PAD
        row = b * V_PAD + v + 2
        slot2 = lax.rem(v + 2, 3)
        pltpu.sync_copy(
            p_hbm.at[pl.ds(pl.multiple_of(row * W_PAD, 8), W_PAD)],
            win.at[pl.ds(pl.multiple_of(slot2 * W_PAD, 8), W_PAD)])
        for i in range(3):
            base_i = lax.rem(v + i, 3) * W_PAD + w0
            for j in range(3):
                chn = 3 * i + j
                for k in range(SP // LANES):
                    idx = base_i + (j + LANES * k) + lane
                    obuf[pl.ds(chn * SP + LANES * k, LANES)] = (
                        plsc.load_gather(win, [idx]))
        for chn in range(9):
            orow = (b * 9 + chn) * VIEW + v
            pltpu.sync_copy(
                obuf.at[pl.ds(chn * SP, SP)],
                out_hbm.at[pl.ds(pl.multiple_of(orow * SP, 8), SP)])
        return carry

    lax.fori_loop(0, 64, body, 0)


def _sc_windows(p_flat, c_flat):
    batch = 4
    return pl.kernel(
        _sc_body,
        out_type=jax.ShapeDtypeStruct((batch * 9 * VIEW * SP,), jnp.float32),
        mesh=plsc.VectorSubcoreMesh(core_axis_name="c", subcore_axis_name="s"),
        compiler_params=pltpu.CompilerParams(needs_layout_passes=False),
        scratch_types=[
            pltpu.VMEM((64,), jnp.int32),
            pltpu.VMEM((3 * W_PAD,), jnp.float32),
            pltpu.VMEM((9 * SP,), jnp.float32),
        ],
    )(p_flat, c_flat)


VB = 16


def _tc_body(c_ref, x_ref, o_ref):
    cf = c_ref[...][:, :, 0:1].astype(jnp.float32)
    inv_det = jnp.float32(1.0 / DET)
    mn = ((cf - 127.5) / DET) * 2.0 - 1.0 - inv_det
    mx = ((cf + 127.5) / DET) * 2.0 - 1.0 + inv_det
    scale_f = 2.0 / (mx - mn)
    mean = (mn + mx) / 2.0
    x = x_ref[...]
    par = lax.broadcasted_iota(jnp.int32, x.shape, 2) & 1
    o_ref[...] = jnp.where(par == 0, scale_f * (x - mean), x)


def _tc_rescale(c_arr, bp3):
    batch, view, wl = bp3.shape
    c_e = jnp.broadcast_to(c_arr.reshape(batch, view, 1).astype(jnp.int32),
                           (batch, view, 8))
    return pl.pallas_call(
        _tc_body,
        grid=(batch, view // VB),
        in_specs=[
            pl.BlockSpec((1, VB, 8), lambda b, v: (b, v, 0)),
            pl.BlockSpec((1, VB, wl), lambda b, v: (b, v, 0)),
        ],
        out_specs=pl.BlockSpec((1, VB, wl), lambda b, v: (b, v, 0)),
        out_shape=jax.ShapeDtypeStruct(bp3.shape, bp3.dtype),
    )(c_e, bp3)


def kernel(sinogram, bp_grid, mask_idx, scale, center):
    batch, _, view, _ = sinogram.shape
    c = center[mask_idx[0], mask_idx[1]] // scale
    c = c.astype(jnp.int32)

    p = jnp.pad(sinogram[:, 0], ((0, 0), (1, 1), (1, 7)))
    sino_out = _sc_windows(p.reshape(-1), c.reshape(-1))
    sino_out = sino_out.reshape(batch, 9, VIEW, SP)

    bp3 = bp_grid.reshape(batch, view, -1)
    bp_out = _tc_rescale(c, bp3).reshape(bp_grid.shape)
    return (sino_out, bp_out)
